# baseline (speedup 1.0000x reference)
import functools

import jax
import jax.numpy as jnp
from jax import lax
from jax.experimental import pallas as pl
from jax.experimental.pallas import tpu as pltpu
from jax.experimental.pallas import tpu_sc as plsc

B, T, D = 64, 2048, 64
V = 1000000
NC, NS = 2, 16
NW = NC * NS
TS = 128
TP = TS // 2
BH = B // NC
BC = 2
NCH = BH // BC
LANES = 16
NQ = D // LANES

CR = 320
NCK = V // CR
KFULL = NCK // NW


def _detile(tok_hbm, out_hbm, buf_v, pair_v, rsem, wsem):
    c = lax.axis_index("c")
    s = lax.axis_index("s")
    w = s * NC + c

    def fire_read(k, slot):
        g = w + k * NW
        r0 = pl.multiple_of(g * CR, 8)
        return pltpu.async_copy(
            tok_hbm.at[pl.ds(r0, CR), :], buf_v.at[slot], rsem
        )

    def repack(slot):
        def body(p, carry):
            for k8 in range(2 * NQ):
                pair_v[slot, p, pl.ds(k8 * LANES, LANES)] = (
                    buf_v[slot, 2 * p + k8 // NQ, pl.ds((k8 % NQ) * LANES, LANES)]
                )
            return carry
        lax.fori_loop(0, CR // 2, body, 0)

    rhandles = {0: fire_read(0, 0), 1: None}
    whandles = {0: None, 1: None}
    for k in range(KFULL):
        slot = k % 2
        if k + 1 < KFULL:
            rhandles[(k + 1) % 2] = fire_read(k + 1, (k + 1) % 2)
        rhandles[slot].wait()
        if whandles[slot] is not None:
            whandles[slot].wait()
        repack(slot)
        g = w + k * NW
        p0 = pl.multiple_of(g * (CR // 2), 8)
        whandles[slot] = pltpu.async_copy(
            pair_v.at[slot], out_hbm.at[pl.ds(p0, CR // 2), :], wsem
        )
    for slot in range(2):
        whandles[slot].wait()

    @pl.when(w < NCK - KFULL * NW)
    def _():
        g = KFULL * NW + w
        r0 = pl.multiple_of(g * CR, 8)
        p0 = pl.multiple_of(g * (CR // 2), 8)
        pltpu.sync_copy(tok_hbm.at[pl.ds(r0, CR), :], buf_v.at[0])
        repack(0)
        pltpu.sync_copy(pair_v.at[0], out_hbm.at[pl.ds(p0, CR // 2), :])


def _run(idx_hbm, tok_hbm, pos_hbm, out_hbm,
         idx_v, pos_v, rows_v, out_v, gsem, osem):
    c = lax.axis_index("c")
    s = lax.axis_index("s")
    b0 = c * BH
    t0 = s * TS
    tp0 = s * TP
    pltpu.sync_copy(idx_hbm.at[pl.ds(b0, BH), pl.ds(t0, TS)], idx_v)
    pltpu.sync_copy(pos_hbm.at[pl.ds(tp0, TP), :], pos_v)

    def fire_gathers(ch, slot):
        return [
            pltpu.async_copy(
                tok_hbm.at[idx_v.at[ch * BC + j]], rows_v.at[slot, j], gsem
            )
            for j in range(BC)
        ]

    ghandles = {0: fire_gathers(0, 0), 1: None}
    ohandles = {0: None, 1: None}

    for ch in range(NCH):
        slot = ch % 2
        if ch + 1 < NCH:
            ghandles[(ch + 1) % 2] = fire_gathers(ch + 1, (ch + 1) % 2)
        for h in ghandles[slot]:
            h.wait()
        if ohandles[slot] is not None:
            ohandles[slot].wait()

        def extract(tl, carry, slot=slot):
            pvs = [pos_v[tl, pl.ds(k * LANES, LANES)] for k in range(2 * NQ)]
            for j in range(BC):
                for e in range(2):
                    tt = tl * 2 + e
                    for q in range(NQ):
                        val = rows_v[slot, j, tt, pl.ds(q * LANES, LANES)]
                        k = e * NQ + q
                        out_v[slot, j, tl, pl.ds(k * LANES, LANES)] = val + pvs[k]
            return carry

        lax.fori_loop(0, TP, extract, 0)
        ohandles[slot] = pltpu.async_copy(
            out_v.at[slot],
            out_hbm.at[pl.ds(b0 + ch * BC, BC), pl.ds(tp0, TP), :],
            osem,
        )
    for slot in range(2):
        if ohandles[slot] is not None:
            ohandles[slot].wait()


def kernel(idx, token_table, position_table):
    mesh = plsc.VectorSubcoreMesh(core_axis_name="c", subcore_axis_name="s")
    detile = functools.partial(
        pl.kernel,
        out_type=jax.ShapeDtypeStruct((V // 2, 2 * D), jnp.float32),
        mesh=mesh,
        compiler_params=pltpu.CompilerParams(use_tc_tiling_on_sc=True),
        scratch_types=[
            pltpu.VMEM((2, CR, D), jnp.float32),
            pltpu.VMEM((2, CR // 2, 2 * D), jnp.float32),
            pltpu.SemaphoreType.DMA,
            pltpu.SemaphoreType.DMA,
        ],
    )(_detile)
    run = functools.partial(
        pl.kernel,
        out_type=jax.ShapeDtypeStruct((B, T // 2, 2 * D), jnp.float32),
        mesh=mesh,
        compiler_params=pltpu.CompilerParams(use_tc_tiling_on_sc=False),
        scratch_types=[
            pltpu.VMEM((BH, TS), jnp.int32),
            pltpu.VMEM((TP, 2 * D), jnp.float32),
            pltpu.VMEM((2, BC, TS, D), jnp.float32),
            pltpu.VMEM((2, BC, TP, 2 * D), jnp.float32),
            pltpu.SemaphoreType.DMA,
            pltpu.SemaphoreType.DMA,
        ],
    )(_run)
    tok_lin = detile(token_table).reshape(V, D)
    out_pair = run(
        idx.astype(jnp.int32),
        tok_lin,
        position_table.reshape(T // 2, 2 * D),
    )
    return out_pair.reshape(B, T, D)


# speedup vs baseline: 1.2985x; 1.2985x over previous
import functools

import jax
import jax.numpy as jnp
from jax import lax
from jax.experimental import pallas as pl
from jax.experimental.pallas import tpu as pltpu
from jax.experimental.pallas import tpu_sc as plsc

B, T, D = 64, 2048, 64
V = 1000000
NC, NS = 2, 16
TS = 128
TP = TS // 2
BH = B // NC
BC = 2
NCH = BH // BC
LANES = 16
NQ = D // LANES


def _run(idx_hbm, tok_hbm, pos_hbm, out_hbm,
         idx_v, pos_v, rows_v, out_v, gsem, osem):
    c = lax.axis_index("c")
    s = lax.axis_index("s")
    b0 = c * BH
    t0 = s * TS
    tp0 = s * TP
    pltpu.sync_copy(idx_hbm.at[pl.ds(b0, BH), pl.ds(t0, TS)], idx_v)
    pltpu.sync_copy(pos_hbm.at[pl.ds(tp0, TP), :], pos_v)

    def fire_gathers(ch, slot):
        return [
            pltpu.async_copy(
                tok_hbm.at[idx_v.at[ch * BC + j]], rows_v.at[slot, j], gsem
            )
            for j in range(BC)
        ]

    ghandles = {0: fire_gathers(0, 0), 1: None}
    ohandles = {0: None, 1: None}

    for ch in range(NCH):
        slot = ch % 2
        if ch + 1 < NCH:
            ghandles[(ch + 1) % 2] = fire_gathers(ch + 1, (ch + 1) % 2)
        for h in ghandles[slot]:
            h.wait()
        if ohandles[slot] is not None:
            ohandles[slot].wait()

        def extract(tl, carry, slot=slot):
            pvs = [pos_v[tl, pl.ds(k * LANES, LANES)] for k in range(2 * NQ)]
            for j in range(BC):
                for e in range(2):
                    tt = tl * 2 + e
                    for q in range(NQ):
                        val = rows_v[slot, j, tt, pl.ds(q * LANES, LANES)]
                        k = e * NQ + q
                        out_v[slot, j, tl, pl.ds(k * LANES, LANES)] = val + pvs[k]
            return carry

        lax.fori_loop(0, TP, extract, 0)
        ohandles[slot] = pltpu.async_copy(
            out_v.at[slot],
            out_hbm.at[pl.ds(b0 + ch * BC, BC), pl.ds(tp0, TP), :],
            osem,
        )
    for slot in range(2):
        if ohandles[slot] is not None:
            ohandles[slot].wait()


def kernel(idx, token_table, position_table):
    mesh = plsc.VectorSubcoreMesh(core_axis_name="c", subcore_axis_name="s")
    run = functools.partial(
        pl.kernel,
        out_type=jax.ShapeDtypeStruct((B, T // 2, 2 * D), jnp.float32),
        mesh=mesh,
        compiler_params=pltpu.CompilerParams(use_tc_tiling_on_sc=True),
        scratch_types=[
            pltpu.VMEM((BH, TS), jnp.int32),
            pltpu.VMEM((TP, 2 * D), jnp.float32),
            pltpu.VMEM((2, BC, TS, 2 * D), jnp.float32),
            pltpu.VMEM((2, BC, TP, 2 * D), jnp.float32),
            pltpu.SemaphoreType.DMA,
            pltpu.SemaphoreType.DMA,
        ],
    )(_run)
    out_pair = run(
        idx.astype(jnp.int32),
        jnp.pad(token_table, ((0, 0), (0, D))),
        position_table.reshape(T // 2, 2 * D),
    )
    return out_pair.reshape(B, T, D)
